# baseline (speedup 1.0000x reference)
import numpy as np
import jax
import jax.numpy as jnp
from jax import lax
from jax.experimental import pallas as pl
from jax.experimental.pallas import tpu as pltpu

_EPS = float(np.finfo(np.float32).eps)
_L1 = 0.1
_L2 = 0.2

_DNT = (((1,), (1,)), ((), ()))


def _round_up(n, m):
    return ((n + m - 1) // m) * m


def _pad_to(a, target_shape):
    pads = [(0, t - s) for s, t in zip(a.shape, target_shape)]
    if all(p == (0, 0) for p in pads):
        return a
    return jnp.pad(a, pads)


def _make_body(n_layers, n_chunks, mc, Cp, c0, l2):
    def body(h0_ref, x_ref, w1_ref, w2m_ref, out_ref, numer_ref):
        for i in range(n_chunks):
            rows = pl.ds(i * mc, mc)
            numer_ref[rows, :] = lax.dot_general(
                x_ref[rows, :], w2m_ref[...], _DNT,
                preferred_element_type=jnp.float32)

        hs = [h0_ref[pl.ds(i * mc, mc), :] for i in range(n_chunks)]
        for l in range(n_layers):
            w1l = w1_ref[l]
            for i in range(n_chunks):
                h = hs[i]
                denom = lax.dot_general(
                    h, w1l, _DNT, preferred_element_type=jnp.float32)
                denom = denom + (l2 * h + c0)
                numer = numer_ref[pl.ds(i * mc, mc), pl.ds(l * Cp, Cp)]
                hs[i] = numer * pl.reciprocal(denom, approx=True) * h
        for i in range(n_chunks):
            out_ref[pl.ds(i * mc, mc), :] = hs[i]
    return body


def kernel(h, x, w1_stack, w2_stack):
    n_layers, comp, features = w2_stack.shape
    batch = h.shape[0]
    out_dtype = h.dtype

    Cp = _round_up(comp, 128)
    Fp = _round_up(features, 128)

    TB = 512
    MC = 128
    Bp = _round_up(batch, TB)
    n_bt = Bp // TB
    n_chunks = TB // MC

    h_p = _pad_to(h.astype(jnp.float32), (Bp, Cp))
    x_p = _pad_to(x.astype(jnp.float32), (Bp, Fp))
    w1_p = _pad_to(w1_stack.astype(jnp.float32), (n_layers, Cp, Cp))
    w2_p = _pad_to(w2_stack.astype(jnp.float32), (n_layers, Cp, Fp))
    w2_m = w2_p.reshape(n_layers * Cp, Fp)

    c0 = _L1 + _EPS
    est = 4 * (TB * (2 * Cp + Fp) + n_layers * Cp * (Cp + Fp)
               + TB * n_layers * Cp)
    kwargs = dict(dimension_semantics=("parallel",))
    if est > 24 * 1024 * 1024:
        kwargs["vmem_limit_bytes"] = int(min(2 * est, 64 * 1024 * 1024))

    out_p = pl.pallas_call(
        _make_body(n_layers, n_chunks, MC, Cp, c0, _L2),
        out_shape=jax.ShapeDtypeStruct((Bp, Cp), jnp.float32),
        grid=(n_bt,),
        in_specs=[
            pl.BlockSpec((TB, Cp), lambda b: (b, 0)),
            pl.BlockSpec((TB, Fp), lambda b: (b, 0)),
            pl.BlockSpec((n_layers, Cp, Cp), lambda b: (0, 0, 0)),
            pl.BlockSpec((n_layers * Cp, Fp), lambda b: (0, 0)),
        ],
        out_specs=pl.BlockSpec((TB, Cp), lambda b: (b, 0)),
        scratch_shapes=[pltpu.VMEM((TB, n_layers * Cp), jnp.float32)],
        compiler_params=pltpu.CompilerParams(**kwargs),
    )(h_p, x_p, w1_p, w2_m)

    return out_p[:batch, :comp].astype(out_dtype)


# speedup vs baseline: 1.3292x; 1.3292x over previous
import numpy as np
import jax
import jax.numpy as jnp
from jax import lax
from jax.experimental import pallas as pl
from jax.experimental.pallas import tpu as pltpu

_EPS = float(np.finfo(np.float32).eps)
_L1 = 0.1
_L2 = 0.2

_DN_NT = (((1,), (0,)), ((), ()))
_DN_TT = (((1,), (1,)), ((), ()))


def _round_up(n, m):
    return ((n + m - 1) // m) * m


def _pad_to(a, target_shape):
    pads = [(0, t - s) for s, t in zip(a.shape, target_shape)]
    if all(p == (0, 0) for p in pads):
        return a
    return jnp.pad(a, pads)


def _make_body(n_layers, n_chunks, lc, Cp, c0, l2):
    def body(h0_ref, x_ref, w1_ref, w2m_ref, out_ref, numer_ref):
        numer_ref[...] = lax.dot_general(
            w2m_ref[...], x_ref[...], _DN_TT,
            preferred_element_type=jnp.float32)

        ht = jnp.swapaxes(h0_ref[...], 0, 1)
        hs = [ht[:, i * lc:(i + 1) * lc] for i in range(n_chunks)]
        for l in range(n_layers):
            w1l = w1_ref[l]
            for i in range(n_chunks):
                h = hs[i]
                denom = lax.dot_general(
                    w1l, h, _DN_NT, preferred_element_type=jnp.float32)
                denom = denom + (l2 * h + c0)
                numer = numer_ref[pl.ds(l * Cp, Cp), pl.ds(i * lc, lc)]
                hs[i] = numer * pl.reciprocal(denom, approx=True) * h
        out_ref[...] = jnp.swapaxes(jnp.concatenate(hs, axis=1), 0, 1)
    return body


def kernel(h, x, w1_stack, w2_stack):
    n_layers, comp, features = w2_stack.shape
    batch = h.shape[0]
    out_dtype = h.dtype

    Cp = _round_up(comp, 128)
    Fp = _round_up(features, 128)

    TB = 512
    LC = 256
    Bp = _round_up(batch, TB)
    n_bt = Bp // TB
    n_chunks = TB // LC

    h_p = _pad_to(h.astype(jnp.float32), (Bp, Cp))
    x_p = _pad_to(x.astype(jnp.float32), (Bp, Fp))
    w1_p = _pad_to(w1_stack.astype(jnp.float32), (n_layers, Cp, Cp))
    w2_p = _pad_to(w2_stack.astype(jnp.float32), (n_layers, Cp, Fp))
    w2_m = w2_p.reshape(n_layers * Cp, Fp)

    c0 = _L1 + _EPS
    est = 4 * (TB * (2 * Cp + Fp) + n_layers * Cp * (Cp + Fp)
               + TB * n_layers * Cp)
    kwargs = dict(dimension_semantics=("parallel",))
    if est > 24 * 1024 * 1024:
        kwargs["vmem_limit_bytes"] = int(min(2 * est, 64 * 1024 * 1024))

    out_p = pl.pallas_call(
        _make_body(n_layers, n_chunks, LC, Cp, c0, _L2),
        out_shape=jax.ShapeDtypeStruct((Bp, Cp), jnp.float32),
        grid=(n_bt,),
        in_specs=[
            pl.BlockSpec((TB, Cp), lambda b: (b, 0)),
            pl.BlockSpec((TB, Fp), lambda b: (b, 0)),
            pl.BlockSpec((n_layers, Cp, Cp), lambda b: (0, 0, 0)),
            pl.BlockSpec((n_layers * Cp, Fp), lambda b: (0, 0)),
        ],
        out_specs=pl.BlockSpec((TB, Cp), lambda b: (b, 0)),
        scratch_shapes=[pltpu.VMEM((n_layers * Cp, TB), jnp.float32)],
        compiler_params=pltpu.CompilerParams(**kwargs),
    )(h_p, x_p, w1_p, w2_m)

    return out_p[:batch, :comp].astype(out_dtype)


# speedup vs baseline: 1.7833x; 1.3417x over previous
import numpy as np
import jax
import jax.numpy as jnp
from jax import lax
from jax.experimental import pallas as pl
from jax.experimental.pallas import tpu as pltpu

_EPS = float(np.finfo(np.float32).eps)
_L1 = 0.1
_L2 = 0.2

_DN = (((1,), (0,)), ((), ()))


def _round_up(n, m):
    return ((n + m - 1) // m) * m


def _pad_to(a, target_shape):
    pads = [(0, t - s) for s, t in zip(a.shape, target_shape)]
    if all(p == (0, 0) for p in pads):
        return a
    return jnp.pad(a, pads)


def _make_body(n_layers, n_chunks, mc, Cp, c0):
    def body(h0_ref, x_ref, w1t_ref, w2t_ref, out_ref, numer_ref):
        for i in range(n_chunks):
            rows = pl.ds(i * mc, mc)
            numer_ref[rows, :] = lax.dot_general(
                x_ref[rows, :], w2t_ref[...], _DN,
                preferred_element_type=jnp.float32)

        hs = [h0_ref[pl.ds(i * mc, mc), :] for i in range(n_chunks)]
        for l in range(n_layers):
            w1l = w1t_ref[l]
            for i in range(n_chunks):
                h = hs[i]
                denom = lax.dot_general(
                    h, w1l, _DN, preferred_element_type=jnp.float32) + c0
                numer = numer_ref[pl.ds(i * mc, mc), pl.ds(l * Cp, Cp)]
                hs[i] = numer * pl.reciprocal(denom, approx=True) * h
        for i in range(n_chunks):
            out_ref[pl.ds(i * mc, mc), :] = hs[i]
    return body


def kernel(h, x, w1_stack, w2_stack):
    n_layers, comp, features = w2_stack.shape
    batch = h.shape[0]
    out_dtype = h.dtype

    Cp = _round_up(comp, 128)
    Fp = _round_up(features, 128)

    TB = 1024
    MC = 256
    Bp = _round_up(batch, TB)
    n_bt = Bp // TB
    n_chunks = TB // MC

    h_p = _pad_to(h.astype(jnp.float32), (Bp, Cp))
    x_p = _pad_to(x.astype(jnp.float32), (Bp, Fp))

    w1t = jnp.swapaxes(w1_stack.astype(jnp.float32), 1, 2)
    w1t = w1t + _L2 * jnp.eye(comp, dtype=jnp.float32)
    w1t_p = _pad_to(w1t, (n_layers, Cp, Cp))

    w2_p = _pad_to(w2_stack.astype(jnp.float32), (n_layers, Cp, Fp))
    w2t_all = jnp.transpose(w2_p, (2, 0, 1)).reshape(Fp, n_layers * Cp)

    c0 = _L1 + _EPS
    est = 4 * (TB * (2 * Cp + Fp) + n_layers * Cp * (Cp + Fp)
               + TB * n_layers * Cp)
    params = pltpu.CompilerParams(
        dimension_semantics=("parallel",),
        vmem_limit_bytes=int(min(3 * est, 100 * 1024 * 1024)),
    )

    out_p = pl.pallas_call(
        _make_body(n_layers, n_chunks, MC, Cp, c0),
        out_shape=jax.ShapeDtypeStruct((Bp, Cp), jnp.float32),
        grid=(n_bt,),
        in_specs=[
            pl.BlockSpec((TB, Cp), lambda b: (b, 0)),
            pl.BlockSpec((TB, Fp), lambda b: (b, 0)),
            pl.BlockSpec((n_layers, Cp, Cp), lambda b: (0, 0, 0)),
            pl.BlockSpec((Fp, n_layers * Cp), lambda b: (0, 0)),
        ],
        out_specs=pl.BlockSpec((TB, Cp), lambda b: (b, 0)),
        scratch_shapes=[pltpu.VMEM((TB, n_layers * Cp), jnp.float32)],
        compiler_params=params,
    )(h_p, x_p, w1t_p, w2t_all)

    return out_p[:batch, :comp].astype(out_dtype)
